# baseline (speedup 1.0000x reference)
import jax
import jax.numpy as jnp
from jax import lax
from jax.experimental import pallas as pl
from jax.experimental.pallas import tpu as pltpu

_B = 16384
_C = 1000
_BC = 2048
_BR = 200


def _onehot_block(ids_ref, o_ref):
    r = pl.program_id(0)
    ids = ids_ref[0]
    in_vocab = (ids >= 0) & (ids < _C)
    mapped = jnp.where(in_vocab, ids, _C - 1)
    row = lax.broadcasted_iota(jnp.int32, (_BR, _BC), 0) + r * _BR
    o_ref[...] = jnp.where(row == mapped, 1.0, 0.0).astype(jnp.float32)


def kernel(user_ids):
    ids = user_ids.astype(jnp.int32).reshape(_B // _BC, 1, _BC)
    out_t = pl.pallas_call(
        _onehot_block,
        grid=(_C // _BR, _B // _BC),
        in_specs=[pl.BlockSpec((1, 1, _BC), lambda r, j: (j, 0, 0))],
        out_specs=pl.BlockSpec((_BR, _BC), lambda r, j: (r, j)),
        out_shape=jax.ShapeDtypeStruct((_C, _B), jnp.float32),
    )(ids)
    return out_t.T
